# baseline (speedup 1.0000x reference)
import functools

import jax
import jax.numpy as jnp
from jax import lax
from jax.experimental import pallas as pl
from jax.experimental.pallas import tpu as pltpu
from jax.experimental.pallas import tpu_sc as plsc

B = 32
P = 1024
K = 10
EPS_BN = 1e-5
NEG = -1e30


def _tc_knn_body(x_ref, sq_ref, wb_ref, idx_ref, va_ref):
    b = pl.program_id(0)
    xb = x_ref[0]
    xbf = xb.astype(jnp.bfloat16)
    va_ref[0] = jnp.dot(xbf, wb_ref[...], preferred_element_type=jnp.float32)

    sq = sq_ref[0, 0]
    g = lax.dot_general(xbf, xbf, (((1,), (1,)), ((), ())),
                        preferred_element_type=jnp.float32)
    neg = (2.0 * g - sq[None, :]) - sq[:, None]
    iota = lax.broadcasted_iota(jnp.int32, (P, P), 0).astype(jnp.float32)
    for t in range(K):
        m = jnp.max(neg, axis=0, keepdims=True)
        cand = jnp.where(neg >= m, iota, jnp.float32(P + 1))
        j = jnp.min(cand, axis=0, keepdims=True)
        idx_ref[0, t] = j[0].astype(jnp.int32) + b * P
        if t < K - 1:
            neg = jnp.where(cand == j, NEG, neg)


def _tc_knn(xb, wb):
    c = xb.shape[-1]
    sq = jnp.sum(xb * xb, axis=-1)[:, None, :]
    idx, va = pl.pallas_call(
        _tc_knn_body,
        grid=(B,),
        in_specs=[
            pl.BlockSpec((1, P, c), lambda b: (b, 0, 0)),
            pl.BlockSpec((1, 1, P), lambda b: (b, 0, 0)),
            pl.BlockSpec((c, 64), lambda b: (0, 0)),
        ],
        out_specs=[
            pl.BlockSpec((1, K, P), lambda b: (b, 0, 0)),
            pl.BlockSpec((1, P, 64), lambda b: (b, 0, 0)),
        ],
        out_shape=[
            jax.ShapeDtypeStruct((B, K, P), jnp.int32),
            jax.ShapeDtypeStruct((B, P, 64), jnp.float32),
        ],
    )(xb, sq, wb)
    return idx, va


def _sc_mesh():
    return plsc.VectorSubcoreMesh(core_axis_name="c", subcore_axis_name="s")


def _sc_gather_body(x_hbm, idx_hbm, xj_hbm, idxv, rows, sem):
    cid = lax.axis_index("c")
    sid = lax.axis_index("s")
    wid = cid * 16 + sid
    pltpu.sync_copy(idx_hbm.at[wid], idxv)
    for t in range(K):
        handles = []
        for jj in range(P // 128):
            h = pltpu.async_copy(
                x_hbm.at[idxv.at[pl.ds(t * P + jj * 128, 128)]],
                rows.at[pl.ds(jj * 128, 128)], sem)
            handles.append(h)
        for h in handles:
            h.wait()
        pltpu.sync_copy(rows, xj_hbm.at[pl.ds((wid * K + t) * P, P)])


def _sc_gather(x, idx_flat):
    c = x.shape[-1]
    kern = functools.partial(
        pl.kernel,
        mesh=_sc_mesh(),
        out_type=jax.ShapeDtypeStruct((B * K * P, c), jnp.float32),
        scratch_types=[
            pltpu.VMEM((K * P,), jnp.int32),
            pltpu.VMEM((P, c), jnp.float32),
            pltpu.SemaphoreType.DMA,
        ],
        compiler_params=pltpu.CompilerParams(use_tc_tiling_on_sc=False),
    )(_sc_gather_body)
    return kern(x, idx_flat)


def _tc_edge_body(xj_ref, x_ref, va_ref, wt_ref, s_ref, beta_ref, bias_ref,
                  f_ref, fmax_ref):
    xb = x_ref[0]
    mdd = None
    for t in range(K):
        dd = (xj_ref[0, t] - xb).astype(jnp.bfloat16)
        acc = jnp.dot(dd, wt_ref[...], preferred_element_type=jnp.float32)
        mdd = acc if mdd is None else jnp.maximum(mdd, acc)
    h = (mdd + va_ref[0] + bias_ref[...][None, :]) * s_ref[...][None, :]
    h = h + beta_ref[...][None, :]
    f = jnp.minimum(jnp.maximum(h, 0.0), 6.0)
    f_ref[0] = f
    fmax_ref[0, 0] = jnp.max(f, axis=0)


def _tc_edge(xj, xb, va, wt, svec, beta, bias):
    c = xb.shape[-1]
    f, fmax = pl.pallas_call(
        _tc_edge_body,
        grid=(B,),
        in_specs=[
            pl.BlockSpec((1, K, P, c), lambda b: (b, 0, 0, 0)),
            pl.BlockSpec((1, P, c), lambda b: (b, 0, 0)),
            pl.BlockSpec((1, P, 64), lambda b: (b, 0, 0)),
            pl.BlockSpec((c, 64), lambda b: (0, 0)),
            pl.BlockSpec((64,), lambda b: (0,)),
            pl.BlockSpec((64,), lambda b: (0,)),
            pl.BlockSpec((64,), lambda b: (0,)),
        ],
        out_specs=[
            pl.BlockSpec((1, P, 64), lambda b: (b, 0, 0)),
            pl.BlockSpec((1, 1, 64), lambda b: (b, 0, 0)),
        ],
        out_shape=[
            jax.ShapeDtypeStruct((B, P, 64), jnp.float32),
            jax.ShapeDtypeStruct((B, 1, 64), jnp.float32),
        ],
    )(xj, xb, va, wt, svec, beta, bias)
    return f, fmax


def _layer(xb, x_gather, wt, wb, svec, beta, bias):
    idx, va = _tc_knn(xb, wb)
    xj = _sc_gather(x_gather, idx.reshape(B, K * P))
    cg = x_gather.shape[-1]
    f, fmax = _tc_edge(xj.reshape(B, K, P, cg),
                       x_gather.reshape(B, P, cg), va, wt, svec, beta, bias)
    return f, fmax.reshape(B, 64)


def _bbox_conv(xf, wt, wb, svec, beta, bias, k):
    n = xf.shape[0]
    sq = jnp.sum(xf * xf, axis=1)
    xbf = xf.astype(jnp.bfloat16)
    g = lax.dot_general(xbf, xbf, (((1,), (1,)), ((), ())),
                        preferred_element_type=jnp.float32)
    neg = (2.0 * g - sq[None, :]) - sq[:, None]
    ci = lax.broadcasted_iota(jnp.int32, (n, n), 0) // 8
    cj = lax.broadcasted_iota(jnp.int32, (n, n), 1) // 8
    neg = jnp.where(ci == cj, neg, NEG)
    va = jnp.dot(xbf, wb, preferred_element_type=jnp.float32)
    iota = lax.broadcasted_iota(jnp.int32, (n, n), 0)
    mdd = jnp.full((n, 64), NEG, jnp.float32)
    for _ in range(k):
        m = jnp.max(neg, axis=0, keepdims=True)
        cand = jnp.where(neg >= m, iota, jnp.int32(n + 1))
        j = jnp.min(cand, axis=0, keepdims=True)
        hit = iota == j
        onehot = jnp.where(hit, 1.0, 0.0)
        sel = lax.dot_general(onehot, xf, (((0,), (0,)), ((), ())),
                              preferred_element_type=jnp.float32,
                              precision=lax.Precision.HIGHEST)
        dd = (sel - xf).astype(jnp.bfloat16)
        acc = jnp.dot(dd, wt, preferred_element_type=jnp.float32)
        mdd = jnp.maximum(mdd, acc)
        neg = jnp.where(hit, NEG, neg)
    h = (mdd + va + bias[None, :]) * svec[None, :] + beta[None, :]
    return jnp.minimum(jnp.maximum(h, 0.0), 6.0)


def _final_body(m1_ref, m2_ref, m3_ref, m4_ref, bbox_ref,
                bwt1_ref, bwb1_ref, bs1_ref, bbe1_ref, bbias1_ref,
                bwt2_ref, bwb2_ref, bs2_ref, bbe2_ref, bbias2_ref,
                bpw_ref, bpb_ref, pw_ref, pb_ref, out_ref):
    xf = bbox_ref[...].reshape(B * 8, 3)
    x1 = _bbox_conv(xf, bwt1_ref[...], bwb1_ref[...], bs1_ref[...],
                    bbe1_ref[...], bbias1_ref[...], 4)
    x2 = _bbox_conv(x1, bwt2_ref[...], bwb2_ref[...], bs2_ref[...],
                    bbe2_ref[...], bbias2_ref[...], 4)
    x12 = jnp.concatenate([x1, x2], axis=1).reshape(B, 8, 128)
    xb2 = jnp.max(x12, axis=1)
    fb = jnp.dot(xb2.astype(jnp.bfloat16), bpw_ref[...],
                 preferred_element_type=jnp.float32)
    fb = jnp.minimum(jnp.maximum(fb + bpb_ref[...][None, :], 0.0), 6.0)
    f = jnp.concatenate([m1_ref[...], m2_ref[...], m3_ref[...], m4_ref[...],
                         fb], axis=1)
    o = jnp.dot(f.astype(jnp.bfloat16), pw_ref[...],
                preferred_element_type=jnp.float32)
    o = jnp.tanh(o + pb_ref[...][None, :])
    nrm = jnp.sqrt(jnp.sum(o * o, axis=1, keepdims=True)) + 1e-9
    out_ref[...] = o / nrm


def _final(m1, m2, m3, m4, bbox, q1, q2, bpw, bpb, pw, pb):
    args = (m1, m2, m3, m4, bbox, *q1, *q2,
            bpw.astype(jnp.bfloat16), bpb, pw.astype(jnp.bfloat16), pb)
    return pl.pallas_call(
        _final_body,
        out_shape=jax.ShapeDtypeStruct((B, 128), jnp.float32),
    )(*args)


def _fold(Wfull, bvec, gamma, beta, c, cpad=None):
    if gamma is None:
        s = jnp.ones_like(bvec)
        be = jnp.zeros_like(bvec)
    else:
        s = gamma / jnp.sqrt(1.0 + EPS_BN)
        be = beta
    wt = Wfull[:c].astype(jnp.bfloat16)
    if cpad is not None and cpad > c:
        wt = jnp.pad(wt, ((0, cpad - c), (0, 0)))
    wb = Wfull[c:].astype(jnp.bfloat16)
    return wt, wb, s, be, bvec


def kernel(x, n_pts, bbox, W1, b1, g1, be1, W2, b2, W3, b3, g3, be3, W4, b4,
           bW1, bb1, bg1, bbe1, bW2, bb2, bg2, bbe2, bPW, bPb, PW, Pb):
    xb = x.reshape(B, P, 3)
    xpad = jnp.pad(x, ((0, 0), (0, 13)))

    wt1, wb1, s1, sb1, bias1 = _fold(W1, b1, g1, be1, 3, cpad=16)
    wt2, wb2, s2, sb2, bias2 = _fold(W2, b2, None, None, 64)
    wt3, wb3, s3, sb3, bias3 = _fold(W3, b3, g3, be3, 64)
    wt4, wb4, s4, sb4, bias4 = _fold(W4, b4, None, None, 64)
    q1 = _fold(bW1, bb1, bg1, bbe1, 3)
    q2 = _fold(bW2, bb2, bg2, bbe2, 64)

    idx, va = _tc_knn(xb, wb1)
    xj = _sc_gather(xpad, idx.reshape(B, K * P))
    f1, max1 = _tc_edge(xj.reshape(B, K, P, 16), xpad.reshape(B, P, 16),
                        va, wt1, s1, sb1, bias1)
    max1 = max1.reshape(B, 64)

    f2, max2 = _layer(f1, f1.reshape(B * P, 64), wt2, wb2, s2, sb2, bias2)
    f3, max3 = _layer(f2, f2.reshape(B * P, 64), wt3, wb3, s3, sb3, bias3)
    _, max4 = _layer(f3, f3.reshape(B * P, 64), wt4, wb4, s4, sb4, bias4)

    return _final(max1, max2, max3, max4, bbox, q1, q2, bPW, bPb, PW, Pb)


# speedup vs baseline: 1.0353x; 1.0353x over previous
import functools

import jax
import jax.numpy as jnp
from jax import lax
from jax.experimental import pallas as pl
from jax.experimental.pallas import tpu as pltpu
from jax.experimental.pallas import tpu_sc as plsc

B = 32
P = 1024
K = 10
EPS_BN = 1e-5
NEG = -1e30


def _tc_knn_body(x_ref, sq_ref, wb_ref, idx_ref, va_ref):
    b = pl.program_id(0)
    xb = x_ref[0]
    xbf = xb.astype(jnp.bfloat16)
    va_ref[0] = jnp.dot(xbf, wb_ref[...], preferred_element_type=jnp.float32)

    sq = sq_ref[0, 0]
    g = lax.dot_general(xbf, xbf, (((1,), (1,)), ((), ())),
                        preferred_element_type=jnp.float32)
    neg = (2.0 * g - sq[None, :]) - sq[:, None]
    iota = lax.broadcasted_iota(jnp.int32, (P, P), 0)
    for t in range(K):
        m = jnp.max(neg, axis=0, keepdims=True)
        cand = jnp.where(neg >= m, iota, jnp.int32(P + 1))
        j = jnp.min(cand, axis=0, keepdims=True)
        idx_ref[0, t] = j[0] + b * P
        if t < K - 1:
            neg = jnp.where(iota == j, NEG, neg)


def _tc_knn(xb, wb):
    nb = xb.shape[0]
    c = xb.shape[-1]
    sq = jnp.sum(xb * xb, axis=-1)[:, None, :]
    idx, va = pl.pallas_call(
        _tc_knn_body,
        grid=(nb,),
        in_specs=[
            pl.BlockSpec((1, P, c), lambda b: (b, 0, 0)),
            pl.BlockSpec((1, 1, P), lambda b: (b, 0, 0)),
            pl.BlockSpec((c, 64), lambda b: (0, 0)),
        ],
        out_specs=[
            pl.BlockSpec((1, K, P), lambda b: (b, 0, 0)),
            pl.BlockSpec((1, P, 64), lambda b: (b, 0, 0)),
        ],
        out_shape=[
            jax.ShapeDtypeStruct((nb, K, P), jnp.int32),
            jax.ShapeDtypeStruct((nb, P, 64), jnp.float32),
        ],
    )(xb, sq, wb)
    return idx, va


def _sc_mesh():
    return plsc.VectorSubcoreMesh(core_axis_name="c", subcore_axis_name="s")


def _sc_gather_body(x_hbm, idx_hbm, xj_hbm, idxv, rows, sem, *, nb):
    cid = lax.axis_index("c")
    sid = lax.axis_index("s")
    wid = cid * 16 + sid
    if nb == 32:
        cloud, lo, npts = wid, 0, P
    else:
        cloud = wid // 2
        lo = (wid % 2) * (P // 2)
        npts = P // 2
    pltpu.sync_copy(idx_hbm.at[cloud], idxv)
    for t in range(K):
        handles = []
        for jj in range(npts // 128):
            h = pltpu.async_copy(
                x_hbm.at[idxv.at[pl.ds(t * P + lo + jj * 128, 128)]],
                rows.at[pl.ds(jj * 128, 128)], sem)
            handles.append(h)
        for h in handles:
            h.wait()
        pltpu.sync_copy(rows, xj_hbm.at[pl.ds((cloud * K + t) * P + lo, npts)])


def _sc_gather(x, idx_flat):
    c = x.shape[-1]
    nb = idx_flat.shape[0]
    npts = P if nb == 32 else P // 2
    kern = functools.partial(
        pl.kernel,
        mesh=_sc_mesh(),
        out_type=jax.ShapeDtypeStruct((nb * K * P, c), jnp.float32),
        scratch_types=[
            pltpu.VMEM((K * P,), jnp.int32),
            pltpu.VMEM((npts, c), jnp.float32),
            pltpu.SemaphoreType.DMA,
        ],
        compiler_params=pltpu.CompilerParams(use_tc_tiling_on_sc=False),
    )(functools.partial(_sc_gather_body, nb=nb))
    return kern(x, idx_flat)


def _tc_edge_body(xj_ref, x_ref, va_ref, wt_ref, s_ref, beta_ref, bias_ref,
                  f_ref, fmax_ref):
    xb = x_ref[0]
    mdd = None
    for t in range(K):
        dd = (xj_ref[0, t] - xb).astype(jnp.bfloat16)
        acc = jnp.dot(dd, wt_ref[...], preferred_element_type=jnp.float32)
        mdd = acc if mdd is None else jnp.maximum(mdd, acc)
    h = (mdd + va_ref[0] + bias_ref[...][None, :]) * s_ref[...][None, :]
    h = h + beta_ref[...][None, :]
    f = jnp.minimum(jnp.maximum(h, 0.0), 6.0)
    f_ref[0] = f
    fmax_ref[0, 0] = jnp.max(f, axis=0)


def _tc_edge(xj, xb, va, wt, svec, beta, bias):
    nb = xb.shape[0]
    c = xb.shape[-1]
    f, fmax = pl.pallas_call(
        _tc_edge_body,
        grid=(nb,),
        in_specs=[
            pl.BlockSpec((1, K, P, c), lambda b: (b, 0, 0, 0)),
            pl.BlockSpec((1, P, c), lambda b: (b, 0, 0)),
            pl.BlockSpec((1, P, 64), lambda b: (b, 0, 0)),
            pl.BlockSpec((c, 64), lambda b: (0, 0)),
            pl.BlockSpec((64,), lambda b: (0,)),
            pl.BlockSpec((64,), lambda b: (0,)),
            pl.BlockSpec((64,), lambda b: (0,)),
        ],
        out_specs=[
            pl.BlockSpec((1, P, 64), lambda b: (b, 0, 0)),
            pl.BlockSpec((1, 1, 64), lambda b: (b, 0, 0)),
        ],
        out_shape=[
            jax.ShapeDtypeStruct((nb, P, 64), jnp.float32),
            jax.ShapeDtypeStruct((nb, 1, 64), jnp.float32),
        ],
    )(xj, xb, va, wt, svec, beta, bias)
    return f, fmax


def _layer(xb, x_gather, wt, wb, svec, beta, bias):
    nb = xb.shape[0]
    idx, va = _tc_knn(xb, wb)
    xj = _sc_gather(x_gather, idx.reshape(nb, K * P))
    cg = x_gather.shape[-1]
    f, fmax = _tc_edge(xj.reshape(nb, K, P, cg),
                       x_gather.reshape(nb, P, cg), va, wt, svec, beta, bias)
    return f, fmax.reshape(nb, 64)


def _bbox_conv(xf, wt, wb, svec, beta, bias, k):
    n = xf.shape[0]
    sq = jnp.sum(xf * xf, axis=1)
    xbf = xf.astype(jnp.bfloat16)
    g = lax.dot_general(xbf, xbf, (((1,), (1,)), ((), ())),
                        preferred_element_type=jnp.float32)
    neg = (2.0 * g - sq[None, :]) - sq[:, None]
    ci = lax.broadcasted_iota(jnp.int32, (n, n), 0) // 8
    cj = lax.broadcasted_iota(jnp.int32, (n, n), 1) // 8
    neg = jnp.where(ci == cj, neg, NEG)
    va = jnp.dot(xbf, wb, preferred_element_type=jnp.float32)
    iota = lax.broadcasted_iota(jnp.int32, (n, n), 0)
    mdd = jnp.full((n, 64), NEG, jnp.float32)
    for _ in range(k):
        m = jnp.max(neg, axis=0, keepdims=True)
        cand = jnp.where(neg >= m, iota, jnp.int32(n + 1))
        j = jnp.min(cand, axis=0, keepdims=True)
        hit = iota == j
        onehot = jnp.where(hit, 1.0, 0.0)
        sel = lax.dot_general(onehot, xf, (((0,), (0,)), ((), ())),
                              preferred_element_type=jnp.float32,
                              precision=lax.Precision.HIGHEST)
        dd = (sel - xf).astype(jnp.bfloat16)
        acc = jnp.dot(dd, wt, preferred_element_type=jnp.float32)
        mdd = jnp.maximum(mdd, acc)
        neg = jnp.where(hit, NEG, neg)
    h = (mdd + va + bias[None, :]) * svec[None, :] + beta[None, :]
    return jnp.minimum(jnp.maximum(h, 0.0), 6.0)


def _final_body(m1_ref, m2_ref, m3_ref, m4_ref, bbox_ref,
                bwt1_ref, bwb1_ref, bs1_ref, bbe1_ref, bbias1_ref,
                bwt2_ref, bwb2_ref, bs2_ref, bbe2_ref, bbias2_ref,
                bpw_ref, bpb_ref, pw_ref, pb_ref, out_ref):
    xf = bbox_ref[...].reshape(B * 8, 3)
    x1 = _bbox_conv(xf, bwt1_ref[...], bwb1_ref[...], bs1_ref[...],
                    bbe1_ref[...], bbias1_ref[...], 4)
    x2 = _bbox_conv(x1, bwt2_ref[...], bwb2_ref[...], bs2_ref[...],
                    bbe2_ref[...], bbias2_ref[...], 4)
    x12 = jnp.concatenate([x1, x2], axis=1).reshape(B, 8, 128)
    xb2 = jnp.max(x12, axis=1)
    fb = jnp.dot(xb2.astype(jnp.bfloat16), bpw_ref[...],
                 preferred_element_type=jnp.float32)
    fb = jnp.minimum(jnp.maximum(fb + bpb_ref[...][None, :], 0.0), 6.0)
    f = jnp.concatenate([m1_ref[...], m2_ref[...], m3_ref[...], m4_ref[...],
                         fb], axis=1)
    o = jnp.dot(f.astype(jnp.bfloat16), pw_ref[...],
                preferred_element_type=jnp.float32)
    o = jnp.tanh(o + pb_ref[...][None, :])
    nrm = jnp.sqrt(jnp.sum(o * o, axis=1, keepdims=True)) + 1e-9
    out_ref[...] = o / nrm


def _final(m1, m2, m3, m4, bbox, q1, q2, bpw, bpb, pw, pb):
    args = (m1, m2, m3, m4, bbox, *q1, *q2,
            bpw.astype(jnp.bfloat16), bpb, pw.astype(jnp.bfloat16), pb)
    return pl.pallas_call(
        _final_body,
        out_shape=jax.ShapeDtypeStruct((B, 128), jnp.float32),
    )(*args)


def _fold(Wfull, bvec, gamma, beta, c, cpad=None):
    if gamma is None:
        s = jnp.ones_like(bvec)
        be = jnp.zeros_like(bvec)
    else:
        s = gamma / jnp.sqrt(1.0 + EPS_BN)
        be = beta
    wt = Wfull[:c].astype(jnp.bfloat16)
    if cpad is not None and cpad > c:
        wt = jnp.pad(wt, ((0, cpad - c), (0, 0)))
    wb = Wfull[c:].astype(jnp.bfloat16)
    return wt, wb, s, be, bvec


def kernel(x, n_pts, bbox, W1, b1, g1, be1, W2, b2, W3, b3, g3, be3, W4, b4,
           bW1, bb1, bg1, bbe1, bW2, bb2, bg2, bbe2, bPW, bPb, PW, Pb):
    xb = x.reshape(B, P, 3)
    xpad = jnp.pad(x, ((0, 0), (0, 13)))

    wt1, wb1, s1, sb1, bias1 = _fold(W1, b1, g1, be1, 3, cpad=16)
    wt2, wb2, s2, sb2, bias2 = _fold(W2, b2, None, None, 64)
    wt3, wb3, s3, sb3, bias3 = _fold(W3, b3, g3, be3, 64)
    wt4, wb4, s4, sb4, bias4 = _fold(W4, b4, None, None, 64)
    q1 = _fold(bW1, bb1, bg1, bbe1, 3)
    q2 = _fold(bW2, bb2, bg2, bbe2, 64)

    hb = B // 2
    maxes = []
    for hh in range(2):
        xb_h = xb[hh * hb:(hh + 1) * hb]
        xpad_h = xpad[hh * hb * P:(hh + 1) * hb * P]
        idx, va = _tc_knn(xb_h, wb1)
        xj = _sc_gather(xpad_h, idx.reshape(hb, K * P))
        f1, m1 = _tc_edge(xj.reshape(hb, K, P, 16), xpad_h.reshape(hb, P, 16),
                          va, wt1, s1, sb1, bias1)
        m1 = m1.reshape(hb, 64)
        f2, m2 = _layer(f1, f1.reshape(hb * P, 64), wt2, wb2, s2, sb2, bias2)
        f3, m3 = _layer(f2, f2.reshape(hb * P, 64), wt3, wb3, s3, sb3, bias3)
        _, m4 = _layer(f3, f3.reshape(hb * P, 64), wt4, wb4, s4, sb4, bias4)
        maxes.append((m1, m2, m3, m4))

    max1, max2, max3, max4 = (jnp.concatenate([maxes[0][i], maxes[1][i]])
                              for i in range(4))
    return _final(max1, max2, max3, max4, bbox, q1, q2, bPW, bPb, PW, Pb)
